# baseline (speedup 1.0000x reference)
import functools

import jax
import jax.numpy as jnp
from jax import lax
from jax.experimental import pallas as pl
from jax.experimental.pallas import tpu as pltpu
from jax.experimental.pallas import tpu_sc as plsc

NUM_CLASSES = 1000000
HIDDEN_SIZE = 32
DROPOUT_PROB = 0.1

BATCH = 16384
NUM_CORES = 2
NUM_SUBCORES = 16
NUM_WORKERS = NUM_CORES * NUM_SUBCORES
ROWS_PER_WORKER = BATCH // NUM_WORKERS
LANES = 16
NB = 8
NSTEP = ROWS_PER_WORKER // (2 * NB)


def _sc_gather(labels, table_t):
    mesh = plsc.VectorSubcoreMesh(core_axis_name="c", subcore_axis_name="s")

    @functools.partial(
        pl.kernel,
        mesh=mesh,
        out_type=jax.ShapeDtypeStruct((BATCH * HIDDEN_SIZE,), jnp.float32),
        compiler_params=pltpu.CompilerParams(
            use_tc_tiling_on_sc=True, needs_layout_passes=False
        ),
        scratch_types=[
            pltpu.VMEM((ROWS_PER_WORKER,), jnp.int32),
            pltpu.VMEM((2 * NB, HIDDEN_SIZE, 128), jnp.float32),
            pltpu.VMEM((ROWS_PER_WORKER * HIDDEN_SIZE,), jnp.float32),
            pltpu.SemaphoreType.DMA,
            pltpu.SemaphoreType.DMA,
        ],
    )
    def k(labels_hbm, table_hbm, out_hbm, idx_v, stage_v, out_v, sem_a, sem_b):
        wid = lax.axis_index("s") * NUM_CORES + lax.axis_index("c")
        base = wid * ROWS_PER_WORKER
        pltpu.sync_copy(labels_hbm.at[pl.ds(base, ROWS_PER_WORKER)], idx_v)
        jlo = lax.iota(jnp.int32, LANES)
        jhi = jlo + LANES

        def fire(v, lo, sem):
            for u in range(lo, lo + NB):
                t = (v[u] // 128) * 128
                pltpu.async_copy(
                    table_hbm.at[:, pl.ds(t, 128)], stage_v.at[u], sem
                )

        def drain(sem):
            for _ in range(NB):
                pltpu.make_async_copy(
                    table_hbm.at[:, pl.ds(0, 128)], stage_v.at[0], sem
                ).wait()

        def extract(step, v, lo):
            r0 = step * 2 * NB
            for u in range(lo, lo + NB):
                l = v[u] - (v[u] // 128) * 128
                lane = jnp.full((LANES,), l, jnp.int32)
                row0 = plsc.load_gather(stage_v.at[u], [jlo, lane])
                row1 = plsc.load_gather(stage_v.at[u], [jhi, lane])
                o = (r0 + u) * HIDDEN_SIZE
                out_v[pl.ds(o, LANES)] = row0
                out_v[pl.ds(o + LANES, LANES)] = row1

        v0 = idx_v[pl.ds(0, 2 * NB)]
        fire(v0, 0, sem_a)
        fire(v0, NB, sem_b)

        def body(step, vprev):
            vcur = idx_v[pl.ds(step * 2 * NB, 2 * NB)]
            drain(sem_a)
            extract(step - 1, vprev, 0)
            fire(vcur, 0, sem_a)
            drain(sem_b)
            extract(step - 1, vprev, NB)
            fire(vcur, NB, sem_b)
            return vcur

        vlast = lax.fori_loop(1, NSTEP, body, v0, unroll=False)
        drain(sem_a)
        extract(jnp.int32(NSTEP - 1), vlast, 0)
        drain(sem_b)
        extract(jnp.int32(NSTEP - 1), vlast, NB)
        pltpu.sync_copy(
            out_v,
            out_hbm.at[pl.ds(base * HIDDEN_SIZE, ROWS_PER_WORKER * HIDDEN_SIZE)],
        )

    return k(labels, table_t)


def kernel(labels, train, embed_table):
    if DROPOUT_PROB > 0:
        drop_key = jax.random.key(1)
        drop_ids = jax.random.uniform(drop_key, (labels.shape[0],)) < DROPOUT_PROB
        train_on = jnp.asarray(train) != 0
        labels = jnp.where(jnp.logical_and(train_on, drop_ids), NUM_CLASSES, labels)
    flat = _sc_gather(labels.astype(jnp.int32), embed_table.T)
    return flat.reshape(BATCH, HIDDEN_SIZE)


# speedup vs baseline: 1.0251x; 1.0251x over previous
import functools

import jax
import jax.numpy as jnp
from jax import lax
from jax.experimental import pallas as pl
from jax.experimental.pallas import tpu as pltpu
from jax.experimental.pallas import tpu_sc as plsc

NUM_CLASSES = 1000000
HIDDEN_SIZE = 32
DROPOUT_PROB = 0.1

BATCH = 16384
NUM_CORES = 2
NUM_SUBCORES = 16
NUM_WORKERS = NUM_CORES * NUM_SUBCORES
ROWS_PER_WORKER = BATCH // NUM_WORKERS
LANES = 16
NBUF = 16


def _sc_gather(labels, table_t):
    mesh = plsc.VectorSubcoreMesh(core_axis_name="c", subcore_axis_name="s")

    @functools.partial(
        pl.kernel,
        mesh=mesh,
        out_type=jax.ShapeDtypeStruct((BATCH * HIDDEN_SIZE,), jnp.float32),
        compiler_params=pltpu.CompilerParams(
            use_tc_tiling_on_sc=True, needs_layout_passes=False
        ),
        scratch_types=[
            pltpu.VMEM((ROWS_PER_WORKER,), jnp.int32),
            pltpu.VMEM((NBUF, HIDDEN_SIZE, 128), jnp.float32),
            pltpu.VMEM((ROWS_PER_WORKER * HIDDEN_SIZE,), jnp.float32),
            pltpu.SemaphoreType.DMA,
        ],
    )
    def k(labels_hbm, table_hbm, out_hbm, idx_v, stage_v, out_v, sem):
        wid = lax.axis_index("s") * NUM_CORES + lax.axis_index("c")
        base = wid * ROWS_PER_WORKER
        pltpu.sync_copy(labels_hbm.at[pl.ds(base, ROWS_PER_WORKER)], idx_v)
        jlo = lax.iota(jnp.int32, LANES)
        jhi = jlo + LANES

        def body(step, carry):
            r0 = step * NBUF
            v = idx_v[pl.ds(r0, NBUF)]
            copies = []
            for u in range(NBUF):
                t = (v[u] // 128) * 128
                copies.append(
                    pltpu.async_copy(
                        table_hbm.at[:, pl.ds(t, 128)],
                        stage_v.at[u],
                        sem,
                    )
                )
            for u in range(NBUF):
                copies[u].wait()
                l = v[u] - (v[u] // 128) * 128
                lane = jnp.full((LANES,), l, jnp.int32)
                row0 = plsc.load_gather(stage_v.at[u], [jlo, lane])
                row1 = plsc.load_gather(stage_v.at[u], [jhi, lane])
                o = (r0 + u) * HIDDEN_SIZE
                out_v[pl.ds(o, LANES)] = row0
                out_v[pl.ds(o + LANES, LANES)] = row1
            return carry

        lax.fori_loop(0, ROWS_PER_WORKER // NBUF, body, 0, unroll=False)
        pltpu.sync_copy(
            out_v,
            out_hbm.at[pl.ds(base * HIDDEN_SIZE, ROWS_PER_WORKER * HIDDEN_SIZE)],
        )

    return k(labels, table_t)


def kernel(labels, train, embed_table):
    if DROPOUT_PROB > 0:
        drop_key = jax.random.key(1)
        drop_ids = jax.random.uniform(drop_key, (labels.shape[0],)) < DROPOUT_PROB
        train_on = jnp.asarray(train) != 0
        labels = jnp.where(jnp.logical_and(train_on, drop_ids), NUM_CLASSES, labels)
    flat = _sc_gather(labels.astype(jnp.int32), embed_table.T)
    return flat.reshape(BATCH, HIDDEN_SIZE)
